# baseline (speedup 1.0000x reference)
import functools

import jax
import jax.numpy as jnp
from jax import lax
from jax.experimental import pallas as pl
from jax.experimental.pallas import tpu as pltpu
from jax.experimental.pallas import tpu_sc as plsc

BATCH = 16384
DIM = 64
LANES = 16
NUM_CORES = 2
NUM_SUBCORES = 16
NUM_WORKERS = NUM_CORES * NUM_SUBCORES
B_PER_W = BATCH // NUM_WORKERS
CHUNK = 128
N_CHUNKS = B_PER_W // CHUNK


def _make_kernel():
    mesh = plsc.VectorSubcoreMesh(core_axis_name="c", subcore_axis_name="s")

    @functools.partial(
        pl.kernel,
        mesh=mesh,
        out_type=jax.ShapeDtypeStruct((BATCH,), jnp.float32),
        compiler_params=pltpu.CompilerParams(use_tc_tiling_on_sc=False),
        scratch_types=[
            pltpu.VMEM((B_PER_W,), jnp.int32),
            pltpu.VMEM((B_PER_W,), jnp.int32),
            pltpu.VMEM((B_PER_W,), jnp.int32),
            pltpu.VMEM((B_PER_W,), jnp.int32),
            pltpu.VMEM((B_PER_W,), jnp.int32),
            pltpu.VMEM((B_PER_W,), jnp.int32),
            pltpu.VMEM((CHUNK, 2 * DIM), jnp.float32),
            pltpu.VMEM((CHUNK, 2 * DIM), jnp.float32),
            pltpu.VMEM((CHUNK, 2 * DIM), jnp.float32),
            pltpu.VMEM((B_PER_W,), jnp.float32),
            pltpu.SemaphoreType.DMA,
        ],
    )
    def distmult(head_hbm, rel_hbm, tail_hbm, ent_hbm, relemb_hbm, out_hbm,
                 hidx, ridx, tidx, hoff, roff, toff,
                 hrows, rrows, trows, scores, sem):
        wid = lax.axis_index("s") * NUM_CORES + lax.axis_index("c")
        base = wid * B_PER_W

        pltpu.sync_copy(head_hbm.at[pl.ds(base, B_PER_W)], hidx)
        pltpu.sync_copy(rel_hbm.at[pl.ds(base, B_PER_W)], ridx)
        pltpu.sync_copy(tail_hbm.at[pl.ds(base, B_PER_W)], tidx)

        def prep(v, carry):
            sl = pl.ds(v * LANES, LANES)
            for idx_ref, off_ref in ((hidx, hoff), (ridx, roff), (tidx, toff)):
                raw = idx_ref[sl]
                idx_ref[sl] = raw >> 1
                off_ref[sl] = (raw & 1) * DIM
            return carry

        lax.fori_loop(0, B_PER_W // LANES, prep, 0)

        lane = lax.iota(jnp.int32, LANES)
        dnums = lax.GatherDimensionNumbers(
            offset_dims=(), collapsed_slice_dims=(0,), start_index_map=(0,))

        def shuffle(v, idx):
            return lax.gather(v, idx[:, None], dnums, slice_sizes=(1,),
                              mode=lax.GatherScatterMode.PROMISE_IN_BOUNDS)

        def lane_sum(v):
            for sh in (8, 4, 2, 1):
                v = v + shuffle(v, lane ^ sh)
            return v

        for c in range(N_CHUNKS):
            csl = pl.ds(c * CHUNK, CHUNK)
            cps = [
                pltpu.async_copy(ent_hbm.at[hidx.at[csl]], hrows, sem),
                pltpu.async_copy(relemb_hbm.at[ridx.at[csl]], rrows, sem),
                pltpu.async_copy(ent_hbm.at[tidx.at[csl]], trows, sem),
            ]
            for cp in cps:
                cp.wait()

            def group(g, carry):
                svec = jnp.zeros((LANES,), jnp.float32)
                gsl = pl.ds(c * CHUNK + g * LANES, LANES)
                ohv = hoff[gsl]
                orv = roff[gsl]
                otv = toff[gsl]
                for j in range(LANES):
                    bb = g * LANES + j
                    oh = ohv[j]
                    orl = orv[j]
                    ot = otv[j]
                    acc = None
                    for dc in range(DIM // LANES):
                        d = dc * LANES
                        prod = (hrows[bb, pl.ds(oh + d, LANES)]
                                * rrows[bb, pl.ds(orl + d, LANES)]
                                * trows[bb, pl.ds(ot + d, LANES)])
                        acc = prod if acc is None else acc + prod
                    svec = jnp.where(lane == j, -lane_sum(acc), svec)
                scores[pl.ds(c * CHUNK + g * LANES, LANES)] = svec
                return carry

            lax.fori_loop(0, CHUNK // LANES, group, 0)

        pltpu.sync_copy(scores, out_hbm.at[pl.ds(base, B_PER_W)])

    return distmult


_distmult = _make_kernel()


@jax.jit
def kernel(head, rel, tail, entity_emb, relation_emb):
    ent2 = entity_emb.reshape(entity_emb.shape[0] // 2, 2 * DIM)
    rel2 = relation_emb.reshape(relation_emb.shape[0] // 2, 2 * DIM)
    return _distmult(head, rel, tail, ent2, rel2)
